# Initial kernel scaffold; baseline (speedup 1.0000x reference)
import jax
import jax.numpy as jnp
from jax.experimental import pallas as pl


def kernel(node_feats, edge_index, edge_weights, W1, W2, W3, a1, g1, b1, a2, g2, b2, a3, g3, b3, p1W, p1b, r1W, r1b, p2W, p2b, r2W, r2b, p3W, p3b, r3W, r3b):
    raise NotImplementedError("write your pallas kernel here")


# speedup vs baseline: 2.9893x; 2.9893x over previous
import functools

import jax
import jax.numpy as jnp
from jax import lax
from jax.experimental import pallas as pl
from jax.experimental.pallas import tpu as pltpu
from jax.experimental.pallas import tpu_sc as plsc

N = 10000
E = 320000
DIN = 128
H = 128
R = 64
EPS = 1e-5

NC = 2
NS = 16
NW = NC * NS
LANES = 16
CH = 80
EPW = E // NW
NCHUNK = EPW // CH
NPAD = 10240
RPT = NPAD // NS
ZR = 128
FV = H // LANES

_MESH = plsc.VectorSubcoreMesh(
    core_axis_name="c", subcore_axis_name="s", num_cores=NC, num_subcores=NS
)


def _leaky(x):
    return jnp.where(x > 0, x, 0.01 * x)


def _deg_body(srcs, dsts, dego_out, degi_out, dego_sp, degi_sp,
              idx_v, ones_v, zb_v):
    cid = lax.axis_index("c")
    sid = lax.axis_index("s")
    wid = cid * NS + sid

    @pl.loop(0, CH)
    def _fill_ones(e):
        ones_v[e, :] = jnp.ones((LANES,), jnp.float32)

    @pl.loop(0, ZR)
    def _fill_zero(r):
        zb_v[r, :] = jnp.zeros((LANES,), jnp.float32)

    for t in range(RPT // ZR):
        r0 = sid * RPT + t * ZR
        pltpu.sync_copy(zb_v, dego_sp.at[pl.ds(r0, ZR)])
        pltpu.sync_copy(zb_v, degi_sp.at[pl.ds(r0, ZR)])
    plsc.subcore_barrier()

    base = wid * EPW

    @pl.loop(0, NCHUNK)
    def _chunk(k):
        eb = base + k * CH
        pltpu.sync_copy(srcs.at[pl.ds(eb, CH)], idx_v)
        pltpu.sync_copy(ones_v, dego_sp.at[idx_v], add=True)
        pltpu.sync_copy(dsts.at[pl.ds(eb, CH)], idx_v)
        pltpu.sync_copy(ones_v, degi_sp.at[idx_v], add=True)

    plsc.subcore_barrier()
    for t in range(RPT // ZR):
        r0 = sid * RPT + t * ZR
        pltpu.sync_copy(dego_sp.at[pl.ds(r0, ZR)], dego_out.at[cid, pl.ds(r0, ZR)])
        pltpu.sync_copy(degi_sp.at[pl.ds(r0, ZR)], degi_out.at[cid, pl.ds(r0, ZR)])


_deg_kernel = pl.kernel(
    _deg_body,
    out_type=[
        jax.ShapeDtypeStruct((NC, NPAD, LANES), jnp.float32),
        jax.ShapeDtypeStruct((NC, NPAD, LANES), jnp.float32),
    ],
    mesh=_MESH,
    scratch_types=[
        pltpu.VMEM_SHARED((NPAD, LANES), jnp.float32),
        pltpu.VMEM_SHARED((NPAD, LANES), jnp.float32),
        pltpu.VMEM((CH,), jnp.int32),
        pltpu.VMEM((CH, LANES), jnp.float32),
        pltpu.VMEM((ZR, LANES), jnp.float32),
    ],
)


def _scatter_body(hs, srcs, dsts, ewb, out, agg_sp,
                  src_v, dst_v, ew_v, rows_v, zb_v, sem):
    cid = lax.axis_index("c")
    sid = lax.axis_index("s")
    wid = cid * NS + sid

    @pl.loop(0, ZR)
    def _fill_zero(r):
        for j in range(FV):
            zb_v[r, pl.ds(j * LANES, LANES)] = jnp.zeros((LANES,), jnp.float32)

    for t in range(RPT // ZR):
        r0 = sid * RPT + t * ZR
        pltpu.sync_copy(zb_v, agg_sp.at[pl.ds(r0, ZR)])
    plsc.subcore_barrier()

    base = wid * EPW

    @pl.loop(0, NCHUNK)
    def _chunk(k):
        eb = base + k * CH
        pltpu.sync_copy(srcs.at[pl.ds(eb, CH)], src_v)
        pltpu.sync_copy(dsts.at[pl.ds(eb, CH)], dst_v)
        pltpu.sync_copy(ewb.at[pl.ds(eb, CH)], ew_v)
        pltpu.async_copy(hs.at[src_v], rows_v, sem).wait()

        @pl.loop(0, CH)
        def _scale(e):
            w = ew_v[e, :]
            for j in range(FV):
                sl = pl.ds(j * LANES, LANES)
                rows_v[e, sl] = rows_v[e, sl] * w

        pltpu.sync_copy(rows_v, agg_sp.at[dst_v], add=True)

    plsc.subcore_barrier()
    for t in range(RPT // ZR):
        r0 = sid * RPT + t * ZR
        pltpu.sync_copy(agg_sp.at[pl.ds(r0, ZR)], out.at[cid, pl.ds(r0, ZR)])


_scatter_kernel = pl.kernel(
    _scatter_body,
    out_type=jax.ShapeDtypeStruct((NC, NPAD, H), jnp.float32),
    mesh=_MESH,
    scratch_types=[
        pltpu.VMEM_SHARED((NPAD, H), jnp.float32),
        pltpu.VMEM((CH,), jnp.int32),
        pltpu.VMEM((CH,), jnp.int32),
        pltpu.VMEM((CH, LANES), jnp.float32),
        pltpu.VMEM((CH, H), jnp.float32),
        pltpu.VMEM((ZR, H), jnp.float32),
        pltpu.SemaphoreType.DMA,
    ],
)


def _inv_sqrt_deg(degs):
    d = (degs[0] + degs[1])[:N]
    d = jnp.max(d, axis=-1, keepdims=True)
    return jnp.where(d > 0, lax.rsqrt(d), 0.0)


def _tc_pre_body(x_ref, w1_ref, dego_ref, hs_ref, ro0_ref):
    x = x_ref[:]
    no = _inv_sqrt_deg(dego_ref[:])
    hs_ref[:] = jnp.dot(x, w1_ref[:], preferred_element_type=jnp.float32) * no
    ro0_ref[:] = _leaky(jnp.mean(x, axis=0, keepdims=True))


_tc_pre = pl.pallas_call(
    _tc_pre_body,
    out_shape=[
        jax.ShapeDtypeStruct((N, H), jnp.float32),
        jax.ShapeDtypeStruct((1, DIN), jnp.float32),
    ],
)


def _tc_mid_body(agg_ref, dego_ref, degi_ref, a_ref, g_ref, b_ref,
                 pw_ref, pb_ref, rw_ref, rb_ref, wn_ref,
                 hsn_ref, ro_ref, mh_ref):
    ni = _inv_sqrt_deg(degi_ref[:])
    y = (agg_ref[0] + agg_ref[1])[:N] * ni
    mu = jnp.mean(y, axis=0, keepdims=True)
    sub = y - a_ref[:] * mu
    var = jnp.mean(sub * sub, axis=0, keepdims=True)
    h = _leaky(g_ref[:] * sub * lax.rsqrt(var + EPS) + b_ref[:])
    phi = _leaky(jnp.dot(h, pw_ref[:], preferred_element_type=jnp.float32)
                 + pb_ref[:])
    ro = _leaky(jnp.dot(jnp.mean(phi, axis=0, keepdims=True), rw_ref[:],
                        preferred_element_type=jnp.float32) + rb_ref[:])
    ro_ref[:] = _leaky(ro)
    mh_ref[:] = _leaky(jnp.mean(h, axis=0, keepdims=True))
    no = _inv_sqrt_deg(dego_ref[:])
    hsn_ref[:] = jnp.dot(h, wn_ref[:], preferred_element_type=jnp.float32) * no


_tc_mid = pl.pallas_call(
    _tc_mid_body,
    out_shape=[
        jax.ShapeDtypeStruct((N, H), jnp.float32),
        jax.ShapeDtypeStruct((1, R), jnp.float32),
        jax.ShapeDtypeStruct((1, H), jnp.float32),
    ],
)


def _tc_last_body(agg_ref, degi_ref, a_ref, g_ref, b_ref,
                  pw_ref, pb_ref, rw_ref, rb_ref,
                  ro_ref, mh_ref):
    ni = _inv_sqrt_deg(degi_ref[:])
    y = (agg_ref[0] + agg_ref[1])[:N] * ni
    mu = jnp.mean(y, axis=0, keepdims=True)
    sub = y - a_ref[:] * mu
    var = jnp.mean(sub * sub, axis=0, keepdims=True)
    h = _leaky(g_ref[:] * sub * lax.rsqrt(var + EPS) + b_ref[:])
    phi = _leaky(jnp.dot(h, pw_ref[:], preferred_element_type=jnp.float32)
                 + pb_ref[:])
    ro = _leaky(jnp.dot(jnp.mean(phi, axis=0, keepdims=True), rw_ref[:],
                        preferred_element_type=jnp.float32) + rb_ref[:])
    ro_ref[:] = _leaky(ro)
    mh_ref[:] = _leaky(jnp.mean(h, axis=0, keepdims=True))


_tc_last = pl.pallas_call(
    _tc_last_body,
    out_shape=[
        jax.ShapeDtypeStruct((1, R), jnp.float32),
        jax.ShapeDtypeStruct((1, H), jnp.float32),
    ],
)


def kernel(node_feats, edge_index, edge_weights, W1, W2, W3,
           a1, g1, b1, a2, g2, b2, a3, g3, b3,
           p1W, p1b, r1W, r1b, p2W, p2b, r2W, r2b, p3W, p3b, r3W, r3b):
    srcs = edge_index[0]
    dsts = edge_index[1]
    row = lambda v: v.reshape(1, -1)

    ewb = jnp.broadcast_to(edge_weights[:, None], (E, LANES))

    dego, degi = _deg_kernel(srcs, dsts)

    hs1, ro0 = _tc_pre(node_feats, W1, dego)
    agg1 = _scatter_kernel(hs1, srcs, dsts, ewb)
    hs2, ro1, mh1 = _tc_mid(agg1, dego, degi, row(a1), row(g1), row(b1),
                            p1W, row(p1b), r1W, row(r1b), W2)
    agg2 = _scatter_kernel(hs2, srcs, dsts, ewb)
    hs3, ro2, mh2 = _tc_mid(agg2, dego, degi, row(a2), row(g2), row(b2),
                            p2W, row(p2b), r2W, row(r2b), W3)
    agg3 = _scatter_kernel(hs3, srcs, dsts, ewb)
    ro3, mh3 = _tc_last(agg3, degi, row(a3), row(g3), row(b3),
                        p3W, row(p3b), r3W, row(r3b))

    return jnp.concatenate([ro0, ro1, mh1, ro2, mh2, ro3, mh3], axis=1)
